# Initial kernel scaffold; baseline (speedup 1.0000x reference)
import jax
import jax.numpy as jnp
from jax.experimental import pallas as pl


def kernel(xyz, features, params):
    raise NotImplementedError("write your pallas kernel here")


# speedup vs baseline: 5.4996x; 5.4996x over previous
import functools

import jax
import jax.numpy as jnp
from jax import lax
from jax.experimental import pallas as pl
from jax.experimental.pallas import tpu as pltpu
from jax.experimental.pallas import tpu_sc as plsc

B = 8
N = 4096
P = 1024
RADII_K = ((0.1, 16), (0.2, 32), (0.4, 64))
CIN = 32
CPAD = 48
EPS = 1e-5
BIG = 1e10


def _fps_body(xt_ref, yt_ref, zt_ref, sx_ref, sy_ref, sz_ref):
    X = xt_ref[...]
    Y = yt_ref[...]
    Z = zt_ref[...]
    lanes = lax.broadcasted_iota(jnp.int32, (B, N), 1)
    slot = lax.broadcasted_iota(jnp.int32, (B, P), 1)

    def step(i, carry):
        dists, far, ax, ay, az = carry
        oh = lanes == far
        cx = jnp.sum(jnp.where(oh, X, 0.0), axis=1, keepdims=True)
        cy = jnp.sum(jnp.where(oh, Y, 0.0), axis=1, keepdims=True)
        cz = jnp.sum(jnp.where(oh, Z, 0.0), axis=1, keepdims=True)
        sel = slot == i
        ax = jnp.where(sel, cx, ax)
        ay = jnp.where(sel, cy, ay)
        az = jnp.where(sel, cz, az)
        dx = X - cx
        dy = Y - cy
        dz = Z - cz
        d = dx * dx + dy * dy + dz * dz
        dists = jnp.minimum(dists, d)
        m = jnp.max(dists, axis=1, keepdims=True)
        far = jnp.min(jnp.where(dists == m, lanes, N), axis=1, keepdims=True)
        return dists, far.astype(jnp.int32), ax, ay, az

    d0 = jnp.full((B, N), BIG, jnp.float32)
    f0 = jnp.zeros((B, 1), jnp.int32)
    a0 = jnp.zeros((B, P), jnp.float32)
    _, _, ax, ay, az = lax.fori_loop(0, P, step, (d0, f0, a0, a0, a0))
    sx_ref[...] = ax
    sy_ref[...] = ay
    sz_ref[...] = az


def _run_fps(xt, yt, zt):
    out = [jax.ShapeDtypeStruct((B, P), jnp.float32)] * 3
    return pl.pallas_call(
        _fps_body,
        out_shape=out,
    )(xt, yt, zt)


_PB = 128


def _ballq_body(xt_ref, yt_ref, zt_ref, sx_ref, sy_ref, sz_ref,
                o1_ref, o2_ref, o3_ref):
    b = pl.program_id(0)
    X = xt_ref[0]
    Y = yt_ref[0]
    Z = zt_ref[0]
    sx = sx_ref[0, 0]
    sy = sy_ref[0, 0]
    sz = sz_ref[0, 0]
    dx = sx - X
    dy = sy - Y
    dz = sz - Z
    d2 = dx * dx + dy * dy + dz * dz
    boff = b * N
    for (radius, K), o_ref in zip(RADII_K, (o1_ref, o2_ref, o3_ref)):
        mask = d2 < radius * radius
        cnt = mask.astype(jnp.float32)
        s = 1
        while s < N:
            cnt = cnt + jnp.concatenate(
                [jnp.zeros((_PB, s), jnp.float32), cnt[:, : N - s]], axis=1)
            s *= 2
        total = cnt[:, N - 1 : N]
        first = jnp.sum((cnt <= 0.0).astype(jnp.float32),
                        axis=1, keepdims=True)
        kiota = lax.broadcasted_iota(jnp.int32, (_PB, K), 1)

        def kbody(k, acc, cnt=cnt, total=total, first=first, kiota=kiota):
            kf = k.astype(jnp.float32)
            c = jnp.sum((cnt <= kf).astype(jnp.float32),
                        axis=1, keepdims=True)
            val = jnp.where(kf < total, c, first)
            return jnp.where(kiota == k, val, acc)

        acc = lax.fori_loop(0, K, kbody, jnp.zeros((_PB, K), jnp.float32))
        o_ref[0] = acc.astype(jnp.int32) + boff


def _run_ballq(xt, yt, zt, sx, sy, sz):
    grid = (B, P // _PB)
    pt_spec = pl.BlockSpec((1, 1, N), lambda b, p: (b, 0, 0))
    ss_spec = pl.BlockSpec((1, 1, _PB, 1), lambda b, p: (b, p, 0, 0))
    outs = [jax.ShapeDtypeStruct((B, P, K), jnp.int32) for _, K in RADII_K]
    out_specs = [pl.BlockSpec((1, _PB, K), lambda b, p: (b, p, 0))
                 for _, K in RADII_K]
    args = ([a.reshape(B, 1, N) for a in (xt, yt, zt)]
            + [s.reshape(B, P // _PB, _PB, 1) for s in (sx, sy, sz)])
    return pl.pallas_call(
        _ballq_body,
        grid=grid,
        in_specs=[pt_spec] * 3 + [ss_spec] * 3,
        out_specs=out_specs,
        out_shape=outs,
    )(*args)


_NC = 2
_NS = 16
_NW = _NC * _NS
_CHUNK = 128


def _make_sc_gather(rows_total):
    per_w = rows_total // _NW
    n_chunks = per_w // _CHUNK
    mesh = plsc.VectorSubcoreMesh(core_axis_name="c", subcore_axis_name="s")

    @functools.partial(
        pl.kernel,
        mesh=mesh,
        compiler_params=pltpu.CompilerParams(use_tc_tiling_on_sc=False),
        out_type=jax.ShapeDtypeStruct((rows_total, CPAD), jnp.float32),
        scratch_types=[
            pltpu.VMEM((_CHUNK,), jnp.int32),
            pltpu.VMEM((_CHUNK, CPAD), jnp.float32),
            pltpu.SemaphoreType.DMA,
        ],
    )
    def gather_k(tbl_hbm, idx_hbm, out_hbm, idx_v, rows_v, sem):
        wid = lax.axis_index("s") * _NC + lax.axis_index("c")
        base = wid * per_w

        def chunk(i, carry):
            s0 = base + i * _CHUNK
            pltpu.sync_copy(idx_hbm.at[pl.ds(s0, _CHUNK)], idx_v)
            pltpu.async_copy(tbl_hbm.at[idx_v], rows_v, sem).wait()
            pltpu.sync_copy(rows_v, out_hbm.at[pl.ds(s0, _CHUNK)])
            return carry

        lax.fori_loop(0, n_chunks, chunk, 0)

    return gather_k


_T = 2048


def _adjust(g, ctr):
    pad = jnp.concatenate(
        [jnp.zeros((_T, CIN), jnp.float32), ctr,
         jnp.zeros((_T, CPAD - CIN - 4), jnp.float32)], axis=1)
    return g - pad


def _dot(x, wt):
    return lax.dot_general(x, wt, (((1,), (0,)), ((), ())),
                           preferred_element_type=jnp.float32,
                           precision=lax.Precision.HIGHEST)


def _stats_body(level, g_ref, ctr_ref, *refs):
    out_ref = refs[-1]
    prm = refs[:-1]
    x = _adjust(g_ref[...], ctr_ref[...])
    o = 0
    for l in range(level):
        wt = prm[o][...]
        bias = prm[o + 1][...]
        a = _dot(x, wt) + bias
        if l == level - 1:
            o += 2
            break
        gamma, beta, mean, rstd = (prm[o + 2][...], prm[o + 3][...],
                                   prm[o + 4][...], prm[o + 5][...])
        x = jax.nn.relu((a - mean) * rstd * gamma + beta)
        o += 6
    s1 = jnp.sum(a, axis=0, keepdims=True)
    s2 = jnp.sum(a * a, axis=0, keepdims=True)

    @pl.when(pl.program_id(0) == 0)
    def _():
        out_ref[...] = jnp.zeros(out_ref.shape, out_ref.dtype)

    out_ref[0:1, :] += s1
    out_ref[1:2, :] += s2


def _final_body(K, g_ref, ctr_ref, *refs):
    out_ref = refs[-1]
    prm = refs[:-1]
    x = _adjust(g_ref[...], ctr_ref[...])
    o = 0
    nl = len(prm) // 6
    for l in range(nl):
        wt, bias, gamma, beta, mean, rstd = (r[...] for r in prm[o:o + 6])
        a = _dot(x, wt) + bias
        x = jax.nn.relu((a - mean) * rstd * gamma + beta)
        o += 6
    c = x.shape[1]
    m = jnp.max(x.reshape(_T // K, K, c), axis=1)
    out_ref[...] = m


def _small(x):
    return x.reshape(1, -1)


def _run_stats(level, g, ctr, prms):
    rows = g.shape[0]
    grid = (rows // _T,)
    in_specs = [pl.BlockSpec((_T, CPAD), lambda r: (r, 0)),
                pl.BlockSpec((_T, 4), lambda r: (r, 0))]
    for p in prms:
        in_specs.append(pl.BlockSpec(p.shape, lambda r: (0, 0)))
    cout = prms[-2].shape[1]
    out_spec = pl.BlockSpec((8, cout), lambda r: (0, 0))
    return pl.pallas_call(
        functools.partial(_stats_body, level),
        grid=grid,
        in_specs=in_specs,
        out_specs=out_spec,
        out_shape=jax.ShapeDtypeStruct((8, cout), jnp.float32),
    )(g, ctr, *prms)


def _run_final(K, g, ctr, prms):
    rows = g.shape[0]
    grid = (rows // _T,)
    in_specs = [pl.BlockSpec((_T, CPAD), lambda r: (r, 0)),
                pl.BlockSpec((_T, 4), lambda r: (r, 0))]
    for p in prms:
        in_specs.append(pl.BlockSpec(p.shape, lambda r: (0, 0)))
    cout = prms[-6].shape[1]
    out_spec = pl.BlockSpec((_T // K, cout), lambda r: (r, 0))
    return pl.pallas_call(
        functools.partial(_final_body, K),
        grid=grid,
        in_specs=in_specs,
        out_specs=out_spec,
        out_shape=jax.ShapeDtypeStruct((rows // K, cout), jnp.float32),
    )(g, ctr, *prms)


def _branch_mlp(g, ctr, layers, K):
    rows = g.shape[0]
    n = jnp.float32(rows)
    wts = []
    for li, (W, bb, gamma, beta) in enumerate(layers):
        wt = W.T
        if li == 0:
            wt = jnp.pad(wt, ((0, CPAD - wt.shape[0]), (0, 0)))
        wts.append(wt)
    known = []
    for li, (W, bb, gamma, beta) in enumerate(layers):
        prms = list(known) + [wts[li], _small(bb)]
        sums = _run_stats(li + 1, g, ctr, prms)
        mean = sums[0:1, :] / n
        var = jnp.maximum(sums[1:2, :] / n - mean * mean, 0.0)
        rstd = lax.rsqrt(var + EPS)
        known += [wts[li], _small(bb), _small(gamma), _small(beta), mean, rstd]
    return _run_final(K, g, ctr, known)


def kernel(xyz, features, params):
    xt = xyz[:, :, 0]
    yt = xyz[:, :, 1]
    zt = xyz[:, :, 2]
    sx, sy, sz = _run_fps(xt, yt, zt)
    xyz_ss = jnp.stack([sx, sy, sz], axis=-1)

    idx1, idx2, idx3 = _run_ballq(xt, yt, zt, sx, sy, sz)

    tbl = jnp.concatenate(
        [features, xyz, jnp.zeros((B, N, CPAD - CIN - 3), jnp.float32)],
        axis=-1).reshape(B * N, CPAD)
    flat = jnp.concatenate([idx1.reshape(-1), idx2.reshape(-1),
                            idx3.reshape(-1)])
    rows_total = flat.shape[0]
    g_all = _make_sc_gather(rows_total)(tbl, flat)

    ctr4 = jnp.concatenate([xyz_ss, jnp.zeros((B, P, 1), jnp.float32)], -1)
    outs = []
    off = 0
    for (radius, K), branch in zip(RADII_K, params):
        rows = B * P * K
        g = lax.slice(g_all, (off, 0), (off + rows, CPAD))
        ctr = jnp.repeat(ctr4.reshape(B * P, 1, 4), K, axis=1).reshape(-1, 4)
        off += rows
        outs.append(_branch_mlp(g, ctr, branch, K))
    feat = jnp.concatenate([o.reshape(B, P, -1) for o in outs], axis=-1)
    return xyz_ss, feat


# speedup vs baseline: 9.0991x; 1.6545x over previous
import functools

import jax
import jax.numpy as jnp
from jax import lax
from jax.experimental import pallas as pl
from jax.experimental.pallas import tpu as pltpu
from jax.experimental.pallas import tpu_sc as plsc

B = 8
N = 4096
P = 1024
RADII_K = ((0.1, 16), (0.2, 32), (0.4, 64))
CIN = 32
CPAD = 48
EPS = 1e-5
BIG = 1e10


def _fps_body(xt_ref, yt_ref, zt_ref, sx_ref, sy_ref, sz_ref):
    X = xt_ref[...]
    Y = yt_ref[...]
    Z = zt_ref[...]
    lanes = lax.broadcasted_iota(jnp.int32, (B, N), 1)
    slot = lax.broadcasted_iota(jnp.int32, (B, P), 1)

    def step(i, carry):
        dists, far, ax, ay, az = carry
        oh = lanes == far
        cx = jnp.sum(jnp.where(oh, X, 0.0), axis=1, keepdims=True)
        cy = jnp.sum(jnp.where(oh, Y, 0.0), axis=1, keepdims=True)
        cz = jnp.sum(jnp.where(oh, Z, 0.0), axis=1, keepdims=True)
        sel = slot == i
        ax = jnp.where(sel, cx, ax)
        ay = jnp.where(sel, cy, ay)
        az = jnp.where(sel, cz, az)
        dx = X - cx
        dy = Y - cy
        dz = Z - cz
        d = dx * dx + dy * dy + dz * dz
        dists = jnp.minimum(dists, d)
        m = jnp.max(dists, axis=1, keepdims=True)
        far = jnp.min(jnp.where(dists == m, lanes, N), axis=1, keepdims=True)
        return dists, far.astype(jnp.int32), ax, ay, az

    d0 = jnp.full((B, N), BIG, jnp.float32)
    f0 = jnp.zeros((B, 1), jnp.int32)
    a0 = jnp.zeros((B, P), jnp.float32)
    _, _, ax, ay, az = lax.fori_loop(0, P, step, (d0, f0, a0, a0, a0))
    sx_ref[...] = ax
    sy_ref[...] = ay
    sz_ref[...] = az


def _run_fps(xt, yt, zt):
    out = [jax.ShapeDtypeStruct((B, P), jnp.float32)] * 3
    return pl.pallas_call(
        _fps_body,
        out_shape=out,
    )(xt, yt, zt)


_PB = 128


def _ballq_body(xt_ref, yt_ref, zt_ref, sx_ref, sy_ref, sz_ref,
                o1_ref, o2_ref, o3_ref):
    b = pl.program_id(0)
    X = xt_ref[0]
    Y = yt_ref[0]
    Z = zt_ref[0]
    sx = sx_ref[0, 0]
    sy = sy_ref[0, 0]
    sz = sz_ref[0, 0]
    dx = sx - X
    dy = sy - Y
    dz = sz - Z
    d2 = dx * dx + dy * dy + dz * dz
    boff = b * N
    for (radius, K), o_ref in zip(RADII_K, (o1_ref, o2_ref, o3_ref)):
        mask = d2 < radius * radius
        cnt = mask.astype(jnp.float32)
        s = 1
        while s < N:
            cnt = cnt + jnp.concatenate(
                [jnp.zeros((_PB, s), jnp.float32), cnt[:, : N - s]], axis=1)
            s *= 2
        total = cnt[:, N - 1 : N]
        first = jnp.sum((cnt <= 0.0).astype(jnp.float32),
                        axis=1, keepdims=True)
        kiota = lax.broadcasted_iota(jnp.int32, (_PB, K), 1)

        def kbody(k, acc, cnt=cnt, total=total, first=first, kiota=kiota):
            kf = k.astype(jnp.float32)
            c = jnp.sum((cnt <= kf).astype(jnp.float32),
                        axis=1, keepdims=True)
            val = jnp.where(kf < total, c, first)
            return jnp.where(kiota == k, val, acc)

        acc = lax.fori_loop(0, K, kbody, jnp.zeros((_PB, K), jnp.float32))
        o_ref[0] = acc.astype(jnp.int32) + boff


def _run_ballq(xt, yt, zt, sx, sy, sz):
    grid = (B, P // _PB)
    pt_spec = pl.BlockSpec((1, 1, N), lambda b, p: (b, 0, 0))
    ss_spec = pl.BlockSpec((1, 1, _PB, 1), lambda b, p: (b, p, 0, 0))
    outs = [jax.ShapeDtypeStruct((B, P, K), jnp.int32) for _, K in RADII_K]
    out_specs = [pl.BlockSpec((1, _PB, K), lambda b, p: (b, p, 0))
                 for _, K in RADII_K]
    args = ([a.reshape(B, 1, N) for a in (xt, yt, zt)]
            + [s.reshape(B, P // _PB, _PB, 1) for s in (sx, sy, sz)])
    return pl.pallas_call(
        _ballq_body,
        grid=grid,
        in_specs=[pt_spec] * 3 + [ss_spec] * 3,
        out_specs=out_specs,
        out_shape=outs,
    )(*args)


_NC = 2
_NS = 16
_NW = _NC * _NS
_CHUNK = 128


def _make_sc_gather(rows_total):
    per_w = rows_total // _NW
    n_chunks = per_w // _CHUNK
    mesh = plsc.VectorSubcoreMesh(core_axis_name="c", subcore_axis_name="s")

    @functools.partial(
        pl.kernel,
        mesh=mesh,
        compiler_params=pltpu.CompilerParams(use_tc_tiling_on_sc=False),
        out_type=jax.ShapeDtypeStruct((rows_total, CPAD), jnp.float32),
        scratch_types=[
            pltpu.VMEM((_CHUNK,), jnp.int32),
            pltpu.VMEM((_CHUNK, CPAD), jnp.float32),
            pltpu.SemaphoreType.DMA,
        ],
    )
    def gather_k(tbl_hbm, idx_hbm, out_hbm, idx_v, rows_v, sem):
        wid = lax.axis_index("s") * _NC + lax.axis_index("c")
        base = wid * per_w

        def chunk(i, carry):
            s0 = base + i * _CHUNK
            pltpu.sync_copy(idx_hbm.at[pl.ds(s0, _CHUNK)], idx_v)
            pltpu.async_copy(tbl_hbm.at[idx_v], rows_v, sem).wait()
            pltpu.sync_copy(rows_v, out_hbm.at[pl.ds(s0, _CHUNK)])
            return carry

        lax.fori_loop(0, n_chunks, chunk, 0)

    return gather_k


_T = 4096


def _adjust(g, ctr):
    pad = jnp.concatenate(
        [jnp.zeros((_T, CIN), jnp.float32), ctr,
         jnp.zeros((_T, CPAD - CIN - 4), jnp.float32)], axis=1)
    return g - pad


def _dot(x, wt):
    return lax.dot_general(x, wt, (((1,), (0,)), ((), ())),
                           preferred_element_type=jnp.float32)


def _norm_params(sums, n):
    mean = sums[0:1, :] / n
    var = jnp.maximum(sums[1:2, :] / n - mean * mean, 0.0)
    rstd = lax.rsqrt(var + EPS)
    return mean, rstd


def _stats_body(level, n, g_ref, ctr_ref, *refs):
    out_ref = refs[-1]
    prm = refs[:-1]
    x = _adjust(g_ref[...], ctr_ref[...])
    o = 0
    for l in range(level):
        wt = prm[o][...]
        bias = prm[o + 1][...]
        a = _dot(x, wt) + bias
        if l == level - 1:
            o += 2
            break
        gamma, beta = prm[o + 2][...], prm[o + 3][...]
        mean, rstd = _norm_params(prm[o + 4][...], n)
        x = jax.nn.relu((a - mean) * rstd * gamma + beta)
        o += 5
    s1 = jnp.sum(a, axis=0, keepdims=True)
    s2 = jnp.sum(a * a, axis=0, keepdims=True)

    @pl.when(pl.program_id(0) == 0)
    def _():
        out_ref[...] = jnp.zeros(out_ref.shape, out_ref.dtype)

    out_ref[0:1, :] += s1
    out_ref[1:2, :] += s2


def _final_body(K, n, g_ref, ctr_ref, *refs):
    out_ref = refs[-1]
    prm = refs[:-1]
    x = _adjust(g_ref[...], ctr_ref[...])
    o = 0
    nl = len(prm) // 5
    for l in range(nl):
        wt, bias, gamma, beta = (r[...] for r in prm[o:o + 4])
        mean, rstd = _norm_params(prm[o + 4][...], n)
        a = _dot(x, wt) + bias
        x = jax.nn.relu((a - mean) * rstd * gamma + beta)
        o += 5
    c = x.shape[1]
    m = jnp.max(x.reshape(_T // K, K, c), axis=1)
    out_ref[...] = m


def _small(x):
    return x.reshape(1, -1)


def _run_stats(level, g, ctr, prms):
    rows = g.shape[0]
    grid = (rows // _T,)
    in_specs = [pl.BlockSpec((_T, CPAD), lambda r: (r, 0)),
                pl.BlockSpec((_T, 4), lambda r: (r, 0))]
    for p in prms:
        in_specs.append(pl.BlockSpec(p.shape, lambda r: (0, 0)))
    cout = prms[-2].shape[1]
    out_spec = pl.BlockSpec((8, cout), lambda r: (0, 0))
    return pl.pallas_call(
        functools.partial(_stats_body, level, float(rows)),
        grid=grid,
        in_specs=in_specs,
        out_specs=out_spec,
        out_shape=jax.ShapeDtypeStruct((8, cout), jnp.float32),
    )(g, ctr, *prms)


def _run_final(K, g, ctr, prms):
    rows = g.shape[0]
    grid = (rows // _T,)
    in_specs = [pl.BlockSpec((_T, CPAD), lambda r: (r, 0)),
                pl.BlockSpec((_T, 4), lambda r: (r, 0))]
    for p in prms:
        in_specs.append(pl.BlockSpec(p.shape, lambda r: (0, 0)))
    cout = prms[-5].shape[1]
    out_spec = pl.BlockSpec((_T // K, cout), lambda r: (r, 0))
    return pl.pallas_call(
        functools.partial(_final_body, K, float(rows)),
        grid=grid,
        in_specs=in_specs,
        out_specs=out_spec,
        out_shape=jax.ShapeDtypeStruct((rows // K, cout), jnp.float32),
    )(g, ctr, *prms)


def _branch_mlp(g, ctr, layers, K):
    rows = g.shape[0]
    wts = []
    for li, (W, bb, gamma, beta) in enumerate(layers):
        wt = W.T
        if li == 0:
            wt = jnp.pad(wt, ((0, CPAD - wt.shape[0]), (0, 0)))
        wts.append(wt)
    known = []
    for li, (W, bb, gamma, beta) in enumerate(layers):
        prms = list(known) + [wts[li], _small(bb)]
        sums = _run_stats(li + 1, g, ctr, prms)
        known += [wts[li], _small(bb), _small(gamma), _small(beta), sums]
    return _run_final(K, g, ctr, known)


def kernel(xyz, features, params):
    xt = xyz[:, :, 0]
    yt = xyz[:, :, 1]
    zt = xyz[:, :, 2]
    sx, sy, sz = _run_fps(xt, yt, zt)
    xyz_ss = jnp.stack([sx, sy, sz], axis=-1)

    idx1, idx2, idx3 = _run_ballq(xt, yt, zt, sx, sy, sz)

    tbl = jnp.concatenate(
        [features, xyz, jnp.zeros((B, N, CPAD - CIN - 3), jnp.float32)],
        axis=-1).reshape(B * N, CPAD)
    flat = jnp.concatenate([idx1.reshape(-1), idx2.reshape(-1),
                            idx3.reshape(-1)])
    rows_total = flat.shape[0]
    g_all = _make_sc_gather(rows_total)(tbl, flat)

    ctr4 = jnp.concatenate([xyz_ss, jnp.zeros((B, P, 1), jnp.float32)], -1)
    outs = []
    off = 0
    for (radius, K), branch in zip(RADII_K, params):
        rows = B * P * K
        g = lax.slice(g_all, (off, 0), (off + rows, CPAD))
        ctr = jnp.repeat(ctr4.reshape(B * P, 1, 4), K, axis=1).reshape(-1, 4)
        off += rows
        outs.append(_branch_mlp(g, ctr, branch, K))
    feat = jnp.concatenate([o.reshape(B, P, -1) for o in outs], axis=-1)
    return xyz_ss, feat
